# Initial kernel scaffold; baseline (speedup 1.0000x reference)
import jax
import jax.numpy as jnp
from jax.experimental import pallas as pl


def kernel(x, edge_index, edge_attr, W_feat, b_feat, W_edge, b_edge, W1, b1, W2, b2, gamma, beta, eps_gin):
    raise NotImplementedError("write your pallas kernel here")


# speedup vs baseline: 2.4436x; 2.4436x over previous
import functools

import jax
import jax.numpy as jnp
from jax import lax
from jax.experimental import pallas as pl
from jax.experimental.pallas import tpu as pltpu
from jax.experimental.pallas import tpu_sc as plsc

N = 10000
DF = 128
DE = 16
DM = 128
L = 3

NC, NS, LANES = 2, 16, 16
GRP = 128
G = 80
HG = 40
EPT = G * GRP
E_PAD = NC * NS * EPT
N_PAD = 10240
RPZ = N_PAD // NS
RPO = N // NS


def _sc_body(h_hbm, srcs_hbm, dsts_hbm, e_hbm, out0_hbm, out1_hbm,
             src_v, dst_v, ebuf0, ebuf1, agg_sh, gsem0, gsem1):
    c = lax.axis_index("c")
    s = lax.axis_index("s")
    t = c * NS + s
    bufs = (ebuf0, ebuf1)
    sems = (gsem0, gsem1)

    zv = jnp.zeros((LANES,), jnp.float32)

    def zrow(r, _):
        for j in range(DM // LANES):
            ebuf0[r, pl.ds(j * LANES, LANES)] = zv
        return 0

    lax.fori_loop(0, GRP, zrow, 0)

    def zcopy(k, _):
        pltpu.sync_copy(ebuf0, agg_sh.at[pl.ds(s * RPZ + k * GRP, GRP)])
        return 0

    lax.fori_loop(0, RPZ // GRP, zcopy, 0)
    plsc.subcore_barrier()

    ebase = t * EPT

    def fetch(ph, g, b):
        pltpu.sync_copy(e_hbm.at[pl.ds(ebase + (ph * HG + g) * GRP, GRP)],
                        bufs[b])
        pltpu.async_copy(h_hbm.at[src_v.at[g]], bufs[b], sems[b], add=True)

    def phase(ph, _):
        pltpu.sync_copy(srcs_hbm.at[t, pl.ds(ph * HG, HG)], src_v)
        pltpu.sync_copy(dsts_hbm.at[t, pl.ds(ph * HG, HG)], dst_v)

        for b in range(2):
            fetch(ph, b, b)

        def pair(q, _):
            for b in range(2):
                g = q * 2 + b
                pltpu.make_async_copy(h_hbm.at[src_v.at[g]], bufs[b],
                                      sems[b]).wait()

                def crow(r, _):
                    for j in range(DM // LANES):
                        sl = pl.ds(j * LANES, LANES)
                        bufs[b][r, sl] = jnp.maximum(bufs[b][r, sl], 0.0)
                    return 0

                lax.fori_loop(0, GRP, crow, 0)
                pltpu.sync_copy(bufs[b], agg_sh.at[dst_v.at[g]], add=True)

                @pl.when(g + 2 < HG)
                def _():
                    fetch(ph, g + 2, b)
            return 0

        lax.fori_loop(0, HG // 2, pair, 0)
        return 0

    lax.fori_loop(0, G // HG, phase, 0)
    plsc.subcore_barrier()

    @pl.when(c == 0)
    def _():
        pltpu.sync_copy(agg_sh.at[pl.ds(s * RPZ, RPZ)],
                        out0_hbm.at[pl.ds(s * RPZ, RPZ)])

    @pl.when(c == 1)
    def _():
        pltpu.sync_copy(agg_sh.at[pl.ds(s * RPZ, RPZ)],
                        out1_hbm.at[pl.ds(s * RPZ, RPZ)])


@functools.cache
def _make_sc_agg():
    return pl.kernel(
        _sc_body,
        out_type=[jax.ShapeDtypeStruct((N_PAD, DM), jnp.float32),
                  jax.ShapeDtypeStruct((N_PAD, DM), jnp.float32)],
        mesh=plsc.VectorSubcoreMesh(core_axis_name="c", subcore_axis_name="s",
                                    num_cores=NC, num_subcores=NS),
        scratch_types=[
            pltpu.VMEM((HG, GRP), jnp.int32),
            pltpu.VMEM((HG, GRP), jnp.int32),
            pltpu.VMEM((GRP, DM), jnp.float32),
            pltpu.VMEM((GRP, DM), jnp.float32),
            pltpu.VMEM_SHARED((N_PAD, DM), jnp.float32),
            pltpu.SemaphoreType.DMA,
            pltpu.SemaphoreType.DMA,
        ],
    )


def _sc_agg(h, src, dst, e):
    return _make_sc_agg()(h, src, dst, e)


def _matmul_bias_body(a_ref, w_ref, b_ref, o_ref):
    o_ref[...] = (jnp.dot(a_ref[...], w_ref[...],
                          preferred_element_type=jnp.float32) + b_ref[...])


def _feat_embed(x, w, b2d):
    rb = 1000
    return pl.pallas_call(
        _matmul_bias_body,
        grid=(N // rb,),
        in_specs=[pl.BlockSpec((rb, DF), lambda i: (i, 0)),
                  pl.BlockSpec((DF, DM), lambda i: (0, 0)),
                  pl.BlockSpec((1, DM), lambda i: (0, 0))],
        out_specs=pl.BlockSpec((rb, DM), lambda i: (i, 0)),
        out_shape=jax.ShapeDtypeStruct((N, DM), jnp.float32),
    )(x, w, b2d)


def _edge_encode(a_pad, w, b2d):
    rb = 4096
    return pl.pallas_call(
        _matmul_bias_body,
        grid=(E_PAD // rb,),
        in_specs=[pl.BlockSpec((rb, DE), lambda i: (i, 0)),
                  pl.BlockSpec((DE, DM), lambda i: (0, 0)),
                  pl.BlockSpec((1, DM), lambda i: (0, 0))],
        out_specs=pl.BlockSpec((rb, DM), lambda i: (i, 0)),
        out_shape=jax.ShapeDtypeStruct((E_PAD, DM), jnp.float32),
    )(a_pad, w, b2d)


def _mlp_body(h_ref, p0_ref, p1_ref, w1_ref, b1_ref, w2_ref, b2_ref, eps_ref,
              z2_ref, s_ref, q_ref):
    z = (1.0 + eps_ref[0, 0]) * h_ref[...] + p0_ref[...] + p1_ref[...]
    z1 = jnp.maximum(jnp.dot(z, w1_ref[...],
                             preferred_element_type=jnp.float32) + b1_ref[...],
                     0.0)
    z2 = jnp.dot(z1, w2_ref[...],
                 preferred_element_type=jnp.float32) + b2_ref[...]
    z2_ref[...] = z2
    s_ref[...] = jnp.sum(z2, axis=0).reshape(1, 1, DM)
    q_ref[...] = jnp.sum(z2 * z2, axis=0).reshape(1, 1, DM)


def _gin_mlp(h, p0, p1, w1, b1_2d, w2, b2_2d, eps_2d):
    rb = 1000
    nb = N // rb
    return pl.pallas_call(
        _mlp_body,
        grid=(nb,),
        in_specs=[pl.BlockSpec((rb, DM), lambda i: (i, 0)),
                  pl.BlockSpec((rb, DM), lambda i: (i, 0)),
                  pl.BlockSpec((rb, DM), lambda i: (i, 0)),
                  pl.BlockSpec((DM, 2 * DM), lambda i: (0, 0)),
                  pl.BlockSpec((1, 2 * DM), lambda i: (0, 0)),
                  pl.BlockSpec((2 * DM, DM), lambda i: (0, 0)),
                  pl.BlockSpec((1, DM), lambda i: (0, 0)),
                  pl.BlockSpec((1, 1), lambda i: (0, 0))],
        out_specs=[pl.BlockSpec((rb, DM), lambda i: (i, 0)),
                   pl.BlockSpec((1, 1, DM), lambda i: (i, 0, 0)),
                   pl.BlockSpec((1, 1, DM), lambda i: (i, 0, 0))],
        out_shape=[jax.ShapeDtypeStruct((N, DM), jnp.float32),
                   jax.ShapeDtypeStruct((nb, 1, DM), jnp.float32),
                   jax.ShapeDtypeStruct((nb, 1, DM), jnp.float32)],
    )(h, p0, p1, w1, b1_2d, w2, b2_2d, eps_2d)


def _bn_body(z2_ref, s_ref, q_ref, g_ref, b_ref, o_ref):
    mu = jnp.sum(s_ref[...], axis=0) * (1.0 / N)
    var = jnp.sum(q_ref[...], axis=0) * (1.0 / N) - mu * mu
    inv = lax.rsqrt(var + 1e-5)
    o_ref[...] = jnp.maximum((z2_ref[...] - mu) * inv * g_ref[...] + b_ref[...],
                             0.0)


def _bn_relu(z2, psum, psq, g2d, b2d):
    rb = 1000
    nb = N // rb
    return pl.pallas_call(
        _bn_body,
        grid=(nb,),
        in_specs=[pl.BlockSpec((rb, DM), lambda i: (i, 0)),
                  pl.BlockSpec((nb, 1, DM), lambda i: (0, 0, 0)),
                  pl.BlockSpec((nb, 1, DM), lambda i: (0, 0, 0)),
                  pl.BlockSpec((1, DM), lambda i: (0, 0)),
                  pl.BlockSpec((1, DM), lambda i: (0, 0))],
        out_specs=pl.BlockSpec((rb, DM), lambda i: (i, 0)),
        out_shape=jax.ShapeDtypeStruct((N, DM), jnp.float32),
    )(z2, psum, psq, g2d, b2d)


def kernel(x, edge_index, edge_attr, W_feat, b_feat, W_edge, b_edge,
           W1, b1, W2, b2, gamma, beta, eps_gin):
    pad = E_PAD - edge_index.shape[1]
    src = jnp.pad(edge_index[0], (0, pad)).reshape(NC * NS, G, GRP)
    dst = jnp.pad(edge_index[1], (0, pad),
                  constant_values=N).reshape(NC * NS, G, GRP)
    a_pad = jnp.pad(edge_attr, ((0, pad), (0, 0)))

    h = _feat_embed(x, W_feat, b_feat.reshape(1, DM))
    es = [_edge_encode(a_pad, W_edge[l], b_edge[l].reshape(1, DM))
          for l in range(L)]
    for l in range(L):
        p0, p1 = _sc_agg(h, src, dst, es[l])
        z2, psum, psq = _gin_mlp(h, p0, p1, W1[l], b1[l].reshape(1, 2 * DM),
                                 W2[l], b2[l].reshape(1, DM),
                                 eps_gin[l].reshape(1, 1))
        h = _bn_relu(z2, psum, psq, gamma[l].reshape(1, DM),
                     beta[l].reshape(1, DM))
    return h
